# baseline (speedup 1.0000x reference)
import functools

import jax
import jax.numpy as jnp
from jax import lax
from jax.experimental import pallas as pl
from jax.experimental.pallas import tpu as pltpu
from jax.experimental.pallas import tpu_sc as plsc

N_TOK = 16384
NE = 8192
D = 256
BM = 512
BMC = 512

SC_NC = 2
SC_NS = 16
SC_NW = SC_NC * SC_NS
SC_CH = 128


def _argmin_body(zn2_ref, z2n_ref, cnt_ref, c2_ref, idx_ref):
    s2 = lax.dot_general(zn2_ref[...], cnt_ref[...], (((1,), (0,)), ((), ())),
                         preferred_element_type=jnp.float32)
    t = (z2n_ref[...] + c2_ref[...]) + s2
    idx_ref[0, 0, :] = jnp.argmin(t, axis=1).astype(jnp.int32)


def _argmin_call(zn2, z2n, cnt, c2):
    nb = zn2.shape[0] // BM
    return pl.pallas_call(
        _argmin_body,
        grid=(nb,),
        in_specs=[
            pl.BlockSpec((BM, D), lambda i: (i, 0)),
            pl.BlockSpec((BM, 1), lambda i: (i, 0)),
            pl.BlockSpec((D, NE), lambda i: (0, 0)),
            pl.BlockSpec((1, NE), lambda i: (0, 0)),
        ],
        out_specs=pl.BlockSpec((1, 1, BM), lambda i: (i, 0, 0)),
        out_shape=jax.ShapeDtypeStruct((nb, 1, BM), jnp.int32),
    )(zn2, z2n, cnt, c2)


def _finalize_body(nbc, z_ref, g_ref, zq_ref, loss_ref):
    i = pl.program_id(0)
    g = g_ref[...]
    n = jnp.sqrt(jnp.sum(g * g, axis=1, keepdims=True))
    zq = g / jnp.maximum(n, 1e-12)
    zq_ref[...] = zq
    dlt = zq - z_ref[...]
    ssq = jnp.sum(dlt * dlt, axis=(0, 1), keepdims=True)
    tot = jnp.where(i == 0, jnp.zeros_like(ssq), loss_ref[...]) + ssq
    loss_ref[...] = jnp.where(i == nbc - 1, tot * (1.25 / (N_TOK * D)), tot)


def _finalize_call(z_flat, gathered):
    nbc = z_flat.shape[0] // BMC
    return pl.pallas_call(
        functools.partial(_finalize_body, nbc),
        grid=(nbc,),
        in_specs=[
            pl.BlockSpec((BMC, D), lambda i: (i, 0)),
            pl.BlockSpec((BMC, D), lambda i: (i, 0)),
        ],
        out_specs=[
            pl.BlockSpec((BMC, D), lambda i: (i, 0)),
            pl.BlockSpec((1, 1), lambda i: (0, 0)),
        ],
        out_shape=[
            jax.ShapeDtypeStruct((z_flat.shape[0], D), jnp.float32),
            jax.ShapeDtypeStruct((1, 1), jnp.float32),
        ],
    )(z_flat, gathered)


def _sc_gather_body(bpw, code_hbm, idx_hbm, out_hbm, idx_v, rows0, rows1, sem):
    wid = lax.axis_index("s") * SC_NC + lax.axis_index("c")
    base = wid * bpw
    nch = bpw // SC_CH
    rows = (rows0, rows1)
    pltpu.sync_copy(idx_hbm.at[pl.ds(base, bpw)], idx_v)

    def _start(c):
        return pltpu.async_copy(
            code_hbm.at[idx_v.at[pl.ds(c * SC_CH, SC_CH)]], rows[c % 2], sem)

    cp = _start(0)
    for c in range(nch):
        cp.wait()
        nxt = _start(c + 1) if c + 1 < nch else None
        pltpu.sync_copy(rows[c % 2], out_hbm.at[pl.ds(base + c * SC_CH, SC_CH)])
        cp = nxt


@functools.cache
def _sc_gather(m):
    mesh = plsc.VectorSubcoreMesh(core_axis_name="c", subcore_axis_name="s")
    return pl.kernel(
        functools.partial(_sc_gather_body, m // SC_NW),
        out_type=jax.ShapeDtypeStruct((m, D), jnp.float32),
        mesh=mesh,
        scratch_types=[
            pltpu.VMEM((m // SC_NW,), jnp.int32),
            pltpu.VMEM((SC_CH, D), jnp.float32),
            pltpu.VMEM((SC_CH, D), jnp.float32),
            pltpu.SemaphoreType.DMA,
        ],
    )


def kernel(z, code):
    z_flat = z.reshape(N_TOK, D)
    zn = z_flat / jnp.maximum(
        jnp.linalg.norm(z_flat, axis=-1, keepdims=True), 1e-12)
    cn = code / jnp.maximum(
        jnp.linalg.norm(code, axis=-1, keepdims=True), 1e-12)
    z2n = jnp.sum(zn ** 2, axis=1, keepdims=True)
    c2 = jnp.sum(cn ** 2, axis=1).reshape(1, NE)
    zn2 = zn * (-2.0)
    cnt = cn.T
    idx = _argmin_call(zn2, z2n, cnt, c2).reshape(N_TOK)
    gathered = _sc_gather(N_TOK)(code, idx)
    zq_flat, loss11 = _finalize_call(z_flat, gathered)
    return (zq_flat.reshape(z.shape), loss11[0, 0], (None, None, idx))


# speedup vs baseline: 1.0308x; 1.0308x over previous
import functools

import jax
import jax.numpy as jnp
from jax import lax
from jax.experimental import pallas as pl
from jax.experimental.pallas import tpu as pltpu
from jax.experimental.pallas import tpu_sc as plsc

N_TOK = 16384
NE = 8192
D = 256
BM = 512
BMC = 1024

SC_NC = 2
SC_NS = 16
SC_NW = SC_NC * SC_NS
SC_CH = 128


def _transpose_body(cn_ref, cnt_ref):
    cnt_ref[...] = cn_ref[...].T


def _transpose_call(cn):
    return pl.pallas_call(
        _transpose_body,
        out_specs=pl.BlockSpec((D, NE), lambda: (0, 0)),
        out_shape=jax.ShapeDtypeStruct((D, NE), jnp.float32),
    )(cn)


def _argmin_body(zn_ref, z2n_ref, cnt_ref, c2_ref, idx_ref):
    s2 = lax.dot_general(zn_ref[...] * (-2.0), cnt_ref[...],
                         (((1,), (0,)), ((), ())),
                         preferred_element_type=jnp.float32)
    t = (z2n_ref[...] + c2_ref[...]) + s2
    idx_ref[0, 0, :] = jnp.argmin(t, axis=1).astype(jnp.int32)


def _argmin_call(zn, z2n, cnt, c2):
    nb = zn.shape[0] // BM
    return pl.pallas_call(
        _argmin_body,
        grid=(nb,),
        in_specs=[
            pl.BlockSpec((BM, D), lambda i: (i, 0)),
            pl.BlockSpec((BM, 1), lambda i: (i, 0)),
            pl.BlockSpec((D, NE), lambda i: (0, 0)),
            pl.BlockSpec((1, NE), lambda i: (0, 0)),
        ],
        out_specs=pl.BlockSpec((1, 1, BM), lambda i: (i, 0, 0)),
        out_shape=jax.ShapeDtypeStruct((nb, 1, BM), jnp.int32),
    )(zn, z2n, cnt, c2)


def _loss_body(nbc, z_ref, g_ref, loss_ref):
    i = pl.program_id(0)
    dlt = g_ref[...] - z_ref[...]
    ssq = jnp.sum(dlt * dlt, axis=(0, 1), keepdims=True)
    tot = jnp.where(i == 0, jnp.zeros_like(ssq), loss_ref[...]) + ssq
    loss_ref[...] = jnp.where(i == nbc - 1, tot * (1.25 / (N_TOK * D)), tot)


def _loss_call(z_flat, gathered):
    nbc = z_flat.shape[0] // BMC
    return pl.pallas_call(
        functools.partial(_loss_body, nbc),
        grid=(nbc,),
        in_specs=[
            pl.BlockSpec((BMC, D), lambda i: (i, 0)),
            pl.BlockSpec((BMC, D), lambda i: (i, 0)),
        ],
        out_specs=pl.BlockSpec((1, 1), lambda i: (0, 0)),
        out_shape=jax.ShapeDtypeStruct((1, 1), jnp.float32),
    )(z_flat, gathered)


def _sc_gather_body(bpw, cn_hbm, idx_hbm, out_hbm, idx_v, rows0, rows1, sem):
    wid = lax.axis_index("s") * SC_NC + lax.axis_index("c")
    base = wid * bpw
    nch = bpw // SC_CH
    rows = (rows0, rows1)
    pltpu.sync_copy(idx_hbm.at[pl.ds(base, bpw)], idx_v)

    def _start(c):
        return pltpu.async_copy(
            cn_hbm.at[idx_v.at[pl.ds(c * SC_CH, SC_CH)]], rows[c % 2], sem)

    cp = _start(0)
    for c in range(nch):
        cp.wait()
        nxt = _start(c + 1) if c + 1 < nch else None
        pltpu.sync_copy(rows[c % 2], out_hbm.at[pl.ds(base + c * SC_CH, SC_CH)])
        cp = nxt


@functools.cache
def _sc_gather(m):
    mesh = plsc.VectorSubcoreMesh(core_axis_name="c", subcore_axis_name="s")
    return pl.kernel(
        functools.partial(_sc_gather_body, m // SC_NW),
        out_type=jax.ShapeDtypeStruct((m, D), jnp.float32),
        mesh=mesh,
        scratch_types=[
            pltpu.VMEM((m // SC_NW,), jnp.int32),
            pltpu.VMEM((SC_CH, D), jnp.float32),
            pltpu.VMEM((SC_CH, D), jnp.float32),
            pltpu.SemaphoreType.DMA,
        ],
    )


def kernel(z, code):
    z_flat = z.reshape(N_TOK, D)
    zn = z_flat / jnp.maximum(
        jnp.linalg.norm(z_flat, axis=-1, keepdims=True), 1e-12)
    cn = code / jnp.maximum(
        jnp.linalg.norm(code, axis=-1, keepdims=True), 1e-12)
    z2n = jnp.sum(zn ** 2, axis=1, keepdims=True)
    c2 = jnp.sum(cn ** 2, axis=1).reshape(1, NE)
    cnt = _transpose_call(cn)
    idx = _argmin_call(zn, z2n, cnt, c2).reshape(N_TOK)
    zq_flat = _sc_gather(N_TOK)(cn, idx)
    loss11 = _loss_call(z_flat, zq_flat)
    return (zq_flat.reshape(z.shape), loss11[0, 0], (None, None, idx))


# speedup vs baseline: 1.0391x; 1.0081x over previous
import functools

import jax
import jax.numpy as jnp
from jax import lax
from jax.experimental import pallas as pl
from jax.experimental.pallas import tpu as pltpu
from jax.experimental.pallas import tpu_sc as plsc

N_TOK = 16384
NE = 8192
D = 256
BM = 512
BMC = 1024

SC_NC = 2
SC_NS = 16
SC_NW = SC_NC * SC_NS
SC_CH = 128


def _transpose_body(cn_ref, cnt_ref):
    cnt_ref[...] = cn_ref[...].T


def _transpose_call(cn):
    return pl.pallas_call(
        _transpose_body,
        out_specs=pl.BlockSpec((D, NE), lambda: (0, 0)),
        out_shape=jax.ShapeDtypeStruct((D, NE), jnp.float32),
    )(cn)


def _argmin_body(z_ref, nrm_ref, z2n_ref, cnt_ref, c2_ref, idx_ref):
    zn = z_ref[...] / nrm_ref[...]
    s2 = lax.dot_general(zn * (-2.0), cnt_ref[...],
                         (((1,), (0,)), ((), ())),
                         preferred_element_type=jnp.float32)
    t = (z2n_ref[...] + c2_ref[...]) + s2
    idx_ref[0, 0, :] = jnp.argmin(t, axis=1).astype(jnp.int32)


def _argmin_call(z_flat, nrm, z2n, cnt, c2):
    nb = z_flat.shape[0] // BM
    return pl.pallas_call(
        _argmin_body,
        grid=(nb,),
        in_specs=[
            pl.BlockSpec((BM, D), lambda i: (i, 0)),
            pl.BlockSpec((BM, 1), lambda i: (i, 0)),
            pl.BlockSpec((BM, 1), lambda i: (i, 0)),
            pl.BlockSpec((D, NE), lambda i: (0, 0)),
            pl.BlockSpec((1, NE), lambda i: (0, 0)),
        ],
        out_specs=pl.BlockSpec((1, 1, BM), lambda i: (i, 0, 0)),
        out_shape=jax.ShapeDtypeStruct((nb, 1, BM), jnp.int32),
    )(z_flat, nrm, z2n, cnt, c2)


def _loss_body(nbc, z_ref, g_ref, loss_ref):
    i = pl.program_id(0)
    dlt = g_ref[...] - z_ref[...]
    ssq = jnp.sum(dlt * dlt, axis=(0, 1), keepdims=True)
    tot = jnp.where(i == 0, jnp.zeros_like(ssq), loss_ref[...]) + ssq
    loss_ref[...] = jnp.where(i == nbc - 1, tot * (1.25 / (N_TOK * D)), tot)


def _loss_call(z_flat, gathered):
    nbc = z_flat.shape[0] // BMC
    return pl.pallas_call(
        functools.partial(_loss_body, nbc),
        grid=(nbc,),
        in_specs=[
            pl.BlockSpec((BMC, D), lambda i: (i, 0)),
            pl.BlockSpec((BMC, D), lambda i: (i, 0)),
        ],
        out_specs=pl.BlockSpec((1, 1), lambda i: (0, 0)),
        out_shape=jax.ShapeDtypeStruct((1, 1), jnp.float32),
    )(z_flat, gathered)


def _sc_gather_body(bpw, cn_hbm, idx_hbm, out_hbm, idx_v, rows0, rows1, sem):
    wid = lax.axis_index("s") * SC_NC + lax.axis_index("c")
    base = wid * bpw
    nch = bpw // SC_CH
    rows = (rows0, rows1)
    pltpu.sync_copy(idx_hbm.at[pl.ds(base, bpw)], idx_v)

    def _start(c):
        return pltpu.async_copy(
            cn_hbm.at[idx_v.at[pl.ds(c * SC_CH, SC_CH)]], rows[c % 2], sem)

    cp = _start(0)
    for c in range(nch):
        cp.wait()
        nxt = _start(c + 1) if c + 1 < nch else None
        pltpu.sync_copy(rows[c % 2], out_hbm.at[pl.ds(base + c * SC_CH, SC_CH)])
        cp = nxt


@functools.cache
def _sc_gather(m):
    mesh = plsc.VectorSubcoreMesh(core_axis_name="c", subcore_axis_name="s")
    return pl.kernel(
        functools.partial(_sc_gather_body, m // SC_NW),
        out_type=jax.ShapeDtypeStruct((m, D), jnp.float32),
        mesh=mesh,
        scratch_types=[
            pltpu.VMEM((m // SC_NW,), jnp.int32),
            pltpu.VMEM((SC_CH, D), jnp.float32),
            pltpu.VMEM((SC_CH, D), jnp.float32),
            pltpu.SemaphoreType.DMA,
        ],
    )


def kernel(z, code):
    z_flat = z.reshape(N_TOK, D)
    nrm = jnp.maximum(
        jnp.linalg.norm(z_flat, axis=-1, keepdims=True), 1e-12)
    cn = code / jnp.maximum(
        jnp.linalg.norm(code, axis=-1, keepdims=True), 1e-12)
    z2n = jnp.sum((z_flat / nrm) ** 2, axis=1, keepdims=True)
    c2 = jnp.sum(cn ** 2, axis=1).reshape(1, NE)
    cnt = _transpose_call(cn)
    idx = _argmin_call(z_flat, nrm, z2n, cnt, c2).reshape(N_TOK)
    zq_flat = _sc_gather(N_TOK)(cn, idx)
    loss11 = _loss_call(z_flat, zq_flat)
    return (zq_flat.reshape(z.shape), loss11[0, 0], (None, None, idx))
